# baseline (speedup 1.0000x reference)
import functools

import jax
import jax.numpy as jnp
from jax import lax
from jax.experimental import pallas as pl
from jax.experimental.pallas import tpu as pltpu
from jax.experimental.pallas import tpu_sc as plsc

H = 128
BE = 2000
NC = 2
NS = 16
NW = NC * NS
SEG = 320
NPAD = NW * SEG
PCH = 8
EBITS = 19
EMASK = (1 << EBITS) - 1


def _vector_mesh():
    return plsc.VectorSubcoreMesh(core_axis_name="c", subcore_axis_name="s")


def _sc_params():
    return pltpu.CompilerParams(needs_layout_passes=False)


def _wid():
    return lax.axis_index("s") * NC + lax.axis_index("c")


def _scalar(v):
    return jnp.max(v)


def _partition(dst):
    E = dst.shape[0]
    CS = E // PCH

    @functools.partial(
        pl.kernel,
        out_type=(jax.ShapeDtypeStruct((NW * PCH * CS,), jnp.int32),
                  jax.ShapeDtypeStruct((NW * 16,), jnp.int32)),
        mesh=_vector_mesh(),
        compiler_params=_sc_params(),
        scratch_types=[pltpu.VMEM((CS,), jnp.int32),
                       pltpu.VMEM((CS + 16,), jnp.int32),
                       pltpu.VMEM((16,), jnp.int32),
                       pltpu.SemaphoreType.DMA],
    )
    def k(dst_hbm, pk_hbm, cnt_hbm, dstbuf, stg, cntv, sem):
        w = _wid()
        lo = w * SEG
        sent = jnp.full((16,), SEG << EBITS, jnp.int32)
        iot = lax.iota(jnp.int32, 16)
        cnts = jnp.zeros((16,), jnp.int32)
        UNR = 4
        for i in range(PCH):
            pltpu.async_copy(dst_hbm.at[pl.ds(i * CS, CS)], dstbuf, sem).wait()

            def step(kk, off):
                ms, pks, cnts = [], [], []
                for u in range(UNR):
                    d = dstbuf[pl.ds(kk * (16 * UNR) + u * 16, 16)]
                    m = (d // SEG) == w
                    eid = ((i * CS + kk * (16 * UNR) + u * 16)
                           + lax.iota(jnp.int32, 16))
                    ms.append(m)
                    pks.append(eid | ((d - lo) << EBITS))
                    cnts.append(_scalar(plsc.all_reduce_population_count(m)))
                for u in range(UNR):
                    plsc.store_compressed(stg.at[pl.ds(off, 16)], pks[u],
                                          mask=ms[u])
                    off = off + cnts[u]
                return off

            off = lax.fori_loop(0, CS // (16 * UNR), step, 0)
            stg[pl.ds(off, 16)] = sent
            offp = (off + 15) & ~15
            cnts = cnts + jnp.where(iot == i, offp, 0)
            pltpu.async_copy(stg.at[pl.ds(0, CS)],
                             pk_hbm.at[pl.ds((w * PCH + i) * CS, CS)],
                             sem).wait()
        cntv[...] = cnts
        pltpu.async_copy(cntv, cnt_hbm.at[pl.ds(w * 16, 16)], sem).wait()

    return k(dst)


def _sc_scatter_max(vals, pk, cnt, h_prev):
    E = vals.shape[0]
    CS = E // PCH
    GC = 128

    @functools.partial(
        pl.kernel,
        out_type=jax.ShapeDtypeStruct((NPAD, H), jnp.float32),
        mesh=_vector_mesh(),
        compiler_params=_sc_params(),
        scratch_types=[pltpu.VMEM((SEG + 8, H), jnp.float32),
                       pltpu.VMEM((SEG, H), jnp.float32),
                       pltpu.VMEM((GC, H), jnp.float32),
                       pltpu.VMEM((GC, H), jnp.float32),
                       pltpu.VMEM((GC,), jnp.int32),
                       pltpu.VMEM((GC,), jnp.int32),
                       pltpu.VMEM((GC,), jnp.int32),
                       pltpu.VMEM((GC,), jnp.int32),
                       pltpu.VMEM((16,), jnp.int32),
                       pltpu.SemaphoreType.DMA,
                       pltpu.SemaphoreType.DMA,
                       pltpu.SemaphoreType.DMA,
                       pltpu.SemaphoreType.DMA,
                       pltpu.SemaphoreType.DMA],
    )
    def k(vals_hbm, pk_hbm, cnt_hbm, hprev_hbm, hout_hbm,
          acc, hbuf, rows0, rows1, pbuf0, pbuf1, eidx0, eidx1, cvbuf,
          sem, sp0, sp1, sg0, sg1):
        w = _wid()
        neg = jnp.full((16,), -jnp.inf, jnp.float32)

        @pl.loop(0, SEG + 1)
        def _(r):
            for c in range(H // 16):
                acc[r, pl.ds(c * 16, 16)] = neg

        pltpu.async_copy(cnt_hbm.at[pl.ds(w * 16, 16)], cvbuf, sem).wait()
        cv = cvbuf[...]

        def rmw(pbuf, rows, rem):
            def group(g, _):
                dlv = pbuf[pl.ds(g * 16, 16)] >> EBITS
                for e in range(16):
                    dl = dlv[e]
                    for c in range(H // 16):
                        sl = pl.ds(c * 16, 16)
                        er = g * 16 + e
                        acc[dl, sl] = jnp.maximum(acc[dl, sl], rows[er, sl])
                return 0
            lax.fori_loop(0, rem // 16, group, 0)

        for i in range(PCH):
            cnt_i = cv[i]
            base = (w * PCH + i) * CS
            nsub = (cnt_i + GC - 1) // GC

            def fire_pk(j, pbuf, sp):
                pltpu.async_copy(
                    pk_hbm.at[pl.ds(base + j * GC, GC)], pbuf, sp)

            def wait_pk(j, pbuf, sp):
                pltpu.make_async_copy(
                    pk_hbm.at[pl.ds(base + j * GC, GC)], pbuf, sp).wait()

            def mkeidx(pbuf, eidx):
                for t in range(GC // 16):
                    sl = pl.ds(t * 16, 16)
                    eidx[sl] = jnp.minimum(pbuf[sl] & EMASK, E - 1)

            @pl.when(nsub > 0)
            def _():
                fire_pk(0, pbuf0, sp0)

            def pair(t, _):
                j0 = 2 * t
                j1 = 2 * t + 1

                @pl.when(j0 < nsub)
                def _():
                    wait_pk(j0, pbuf0, sp0)
                    mkeidx(pbuf0, eidx0)
                    pltpu.async_copy(vals_hbm.at[eidx0], rows0, sg0)

                @pl.when(j1 < nsub)
                def _():
                    fire_pk(j1, pbuf1, sp1)

                @pl.when(j0 < nsub)
                def _():
                    pltpu.make_async_copy(vals_hbm.at[eidx0], rows0,
                                          sg0).wait()
                    rmw(pbuf0, rows0, jnp.minimum(GC, cnt_i - j0 * GC))

                @pl.when(j1 < nsub)
                def _():
                    wait_pk(j1, pbuf1, sp1)
                    mkeidx(pbuf1, eidx1)
                    pltpu.async_copy(vals_hbm.at[eidx1], rows1, sg1)

                @pl.when(j0 + 2 < nsub)
                def _():
                    fire_pk(j0 + 2, pbuf0, sp0)

                @pl.when(j1 < nsub)
                def _():
                    pltpu.make_async_copy(vals_hbm.at[eidx1], rows1,
                                          sg1).wait()
                    rmw(pbuf1, rows1, jnp.minimum(GC, cnt_i - j1 * GC))

                return 0

            lax.fori_loop(0, (nsub + 1) // 2, pair, 0)

        pltpu.async_copy(hprev_hbm.at[pl.ds(w * SEG, SEG)], hbuf, sem).wait()

        @pl.loop(0, SEG)
        def _(r):
            for c in range(H // 16):
                sl = pl.ds(c * 16, 16)
                a = acc[r, sl]
                hbuf[r, sl] = hbuf[r, sl] + jnp.where(a == -jnp.inf, 0.0, a)

        pltpu.async_copy(hbuf, hout_hbm.at[pl.ds(w * SEG, SEG)], sem).wait()

    return k(vals, pk, cnt, h_prev)


def _sc_gather(h_pad, dst):
    E = dst.shape[0]
    GC = 128
    per_w = E // NW
    nfull = per_w // GC
    tail = per_w - nfull * GC

    npair = nfull // 2

    @functools.partial(
        pl.kernel,
        out_type=jax.ShapeDtypeStruct((E, H), jnp.float32),
        mesh=_vector_mesh(),
        scratch_types=[pltpu.VMEM((GC,), jnp.int32),
                       pltpu.VMEM((GC,), jnp.int32),
                       pltpu.VMEM((tail,), jnp.int32),
                       pltpu.VMEM((GC, H), jnp.float32),
                       pltpu.VMEM((GC, H), jnp.float32),
                       pltpu.VMEM((tail, H), jnp.float32),
                       pltpu.SemaphoreType.DMA,
                       pltpu.SemaphoreType.DMA,
                       pltpu.SemaphoreType.DMA,
                       pltpu.SemaphoreType.DMA,
                       pltpu.SemaphoreType.DMA,
                       pltpu.SemaphoreType.DMA],
    )
    def k(h_hbm, dst_hbm, out_hbm, idx0, idx1, idxt, rows0, rows1, rowst,
          si0, si1, sg0, sg1, so0, so1):
        base = _wid() * per_w

        def fire_idx(j, idx, si):
            pltpu.async_copy(dst_hbm.at[pl.ds(base + j * GC, GC)], idx, si)

        def wait_idx(j, idx, si):
            pltpu.make_async_copy(dst_hbm.at[pl.ds(base + j * GC, GC)],
                                  idx, si).wait()

        fire_idx(0, idx0, si0)
        fire_idx(1, idx1, si1)

        def pair(t, _):
            j0 = 2 * t
            j1 = 2 * t + 1

            @pl.when(t > 0)
            def _():
                pltpu.make_async_copy(
                    rows0, out_hbm.at[pl.ds(base + (j0 - 2) * GC, GC)],
                    so0).wait()
                pltpu.make_async_copy(
                    rows1, out_hbm.at[pl.ds(base + (j1 - 2) * GC, GC)],
                    so1).wait()

            wait_idx(j0, idx0, si0)
            wait_idx(j1, idx1, si1)
            pltpu.async_copy(h_hbm.at[idx0], rows0, sg0)
            pltpu.async_copy(h_hbm.at[idx1], rows1, sg1)

            pltpu.make_async_copy(h_hbm.at[idx0], rows0, sg0).wait()
            pltpu.async_copy(rows0, out_hbm.at[pl.ds(base + j0 * GC, GC)],
                             so0)
            pltpu.make_async_copy(h_hbm.at[idx1], rows1, sg1).wait()
            pltpu.async_copy(rows1, out_hbm.at[pl.ds(base + j1 * GC, GC)],
                             so1)

            @pl.when(t + 1 < npair)
            def _():
                fire_idx(j0 + 2, idx0, si0)
                fire_idx(j1 + 2, idx1, si1)

            return 0

        lax.fori_loop(0, npair, pair, 0)
        pltpu.make_async_copy(
            rows0, out_hbm.at[pl.ds(base + (nfull - 2) * GC, GC)],
            so0).wait()
        pltpu.make_async_copy(
            rows1, out_hbm.at[pl.ds(base + (nfull - 1) * GC, GC)],
            so1).wait()

        if tail:
            o = base + nfull * GC
            pltpu.async_copy(dst_hbm.at[pl.ds(o, tail)], idxt, si0).wait()
            pltpu.async_copy(h_hbm.at[idxt], rowst, sg0).wait()
            pltpu.async_copy(rowst, out_hbm.at[pl.ds(o, tail)], so0).wait()

    return k(h_pad, dst)


def _b2(b):
    return b.reshape(1, -1)


def _embed_fx_body(ea_raw_ref, w1, b1, w2, b2, f1, c1, f2, c2,
                   ea_out, vals_out):
    x = ea_raw_ref[...]
    t = jnp.maximum(jnp.dot(x, w1[...], preferred_element_type=jnp.float32)
                    + b1[...], 0.0)
    ea = jnp.dot(t, w2[...], preferred_element_type=jnp.float32) + b2[...]
    u = jnp.maximum(jnp.dot(ea, f1[...], preferred_element_type=jnp.float32)
                    + c1[...], 0.0)
    vals = jnp.dot(u, f2[...], preferred_element_type=jnp.float32) + c2[...]
    if ea_out is not None:
        ea_out[...] = ea
    vals_out[...] = vals


def _embed_fx(ea_raw, emb, fx, want_ea):
    E = ea_raw.shape[0]
    grid = (E // BE,)
    w_spec = pl.BlockSpec((H, H), lambda i: (0, 0))
    b_spec = pl.BlockSpec((1, H), lambda i: (0, 0))
    in_specs = [
        pl.BlockSpec((BE, 16), lambda i: (i, 0)),
        pl.BlockSpec((16, H), lambda i: (0, 0)), b_spec, w_spec, b_spec,
        w_spec, b_spec, w_spec, b_spec,
    ]
    out_spec = pl.BlockSpec((BE, H), lambda i: (i, 0))
    if want_ea:
        out_shape = (jax.ShapeDtypeStruct((E, H), jnp.float32),
                     jax.ShapeDtypeStruct((E, H), jnp.float32))
        fn = pl.pallas_call(
            lambda *refs: _embed_fx_body(*refs[:9], refs[9], refs[10]),
            grid=grid, in_specs=in_specs,
            out_specs=(out_spec, out_spec), out_shape=out_shape)
    else:
        out_shape = jax.ShapeDtypeStruct((E, H), jnp.float32)
        fn = pl.pallas_call(
            lambda *refs: _embed_fx_body(*refs[:9], None, refs[9]),
            grid=grid, in_specs=in_specs,
            out_specs=out_spec, out_shape=out_shape)
    return fn(ea_raw, emb["W1"], _b2(emb["b1"]), emb["W2"], _b2(emb["b2"]),
              fx["W1"], _b2(fx["b1"]), fx["W2"], _b2(fx["b2"]))


def _update_fx_body(ea_ref, hg_ref, m1a, m1b, bm1, m2, bm2, f1, c1, f2, c2,
                    ea_out, vals_out):
    ea = ea_ref[...]
    hg = hg_ref[...]
    t = jnp.maximum(
        jnp.dot(ea, m1a[...], preferred_element_type=jnp.float32)
        + jnp.dot(hg, m1b[...], preferred_element_type=jnp.float32)
        + bm1[...], 0.0)
    ea2 = ea + jnp.dot(t, m2[...], preferred_element_type=jnp.float32) + bm2[...]
    u = jnp.maximum(jnp.dot(ea2, f1[...], preferred_element_type=jnp.float32)
                    + c1[...], 0.0)
    vals = jnp.dot(u, f2[...], preferred_element_type=jnp.float32) + c2[...]
    if ea_out is not None:
        ea_out[...] = ea2
    vals_out[...] = vals


def _update_fx(ea, hg, mlp, fx, want_ea):
    E = ea.shape[0]
    grid = (E // BE,)
    w_spec = pl.BlockSpec((H, H), lambda i: (0, 0))
    b_spec = pl.BlockSpec((1, H), lambda i: (0, 0))
    e_spec = pl.BlockSpec((BE, H), lambda i: (i, 0))
    in_specs = [e_spec, e_spec,
                w_spec, w_spec, b_spec, w_spec, b_spec,
                w_spec, b_spec, w_spec, b_spec]
    m1a = mlp["W1"][:H]
    m1b = mlp["W1"][H:]
    if want_ea:
        out_shape = (jax.ShapeDtypeStruct((E, H), jnp.float32),
                     jax.ShapeDtypeStruct((E, H), jnp.float32))
        fn = pl.pallas_call(
            lambda *refs: _update_fx_body(*refs[:11], refs[11], refs[12]),
            grid=grid, in_specs=in_specs,
            out_specs=(e_spec, e_spec), out_shape=out_shape)
    else:
        out_shape = jax.ShapeDtypeStruct((E, H), jnp.float32)
        fn = pl.pallas_call(
            lambda *refs: _update_fx_body(*refs[:11], None, refs[11]),
            grid=grid, in_specs=in_specs,
            out_specs=e_spec, out_shape=out_shape)
    return fn(ea, hg, m1a, m1b, _b2(mlp["b1"]), mlp["W2"], _b2(mlp["b2"]),
              fx["W1"], _b2(fx["b1"]), fx["W2"], _b2(fx["b2"]))


def _field_body(vec_ref, act_ref, w1v, w1a, b1, w2, b2, out_ref):
    t = jnp.maximum(
        jnp.dot(vec_ref[...], w1v[...], preferred_element_type=jnp.float32)
        + jnp.dot(act_ref[...], w1a[...], preferred_element_type=jnp.float32)
        + b1[...], 0.0)
    out_ref[...] = jnp.dot(t, w2[...], preferred_element_type=jnp.float32) + b2[...]


def _field(vec, action, p):
    n = vec.shape[0]
    w1v = p["W1"][:H]
    w1a = p["W1"][H:]
    fn = pl.pallas_call(
        _field_body,
        out_shape=jax.ShapeDtypeStruct((n, 1), jnp.float32))
    return fn(vec, action, w1v, w1a, _b2(p["b1"]), p["W2"],
              _b2(p["b2"])).squeeze(-1)


def _process_type(ea_raw, dst, p):
    n_layers = len(p["layers"])
    pk, cnt = _partition(dst)
    want_ea = n_layers > 1
    res = _embed_fx(ea_raw, p["embed"], p["layers"][0]["fx"], want_ea)
    if want_ea:
        ea, vals = res
    else:
        vals = res
    h = _sc_scatter_max(vals, pk, cnt,
                        jnp.zeros((NPAD, H), jnp.float32))
    for li in range(1, n_layers):
        hg = _sc_gather(h, dst)
        last = li == n_layers - 1
        res = _update_fx(ea, hg, p["layers"][li - 1]["mlp"],
                         p["layers"][li]["fx"], not last)
        if not last:
            ea, vals = res
        else:
            vals = res
        h = _sc_scatter_max(vals, pk, cnt, h)
    return h


def kernel(x_obstacle, x_agent, x_goal, ei_ona, ei_ana, ei_tow,
           ea_ona, ea_ana, ea_tow, action, params):
    n = x_agent.shape[0]
    h_ona = _process_type(ea_ona, ei_ona[1].astype(jnp.int32), params["ona"])
    h_ana = _process_type(ea_ana, ei_ana[1].astype(jnp.int32), params["ana"])
    h_tow = _process_type(ea_tow, ei_tow[1].astype(jnp.int32), params["tow"])
    vec = jnp.maximum(jnp.maximum(jnp.maximum(h_ona, h_ana), h_tow),
                      0.0)[:n]
    return _field(vec, action, params["field"])


# speedup vs baseline: 1.7939x; 1.7939x over previous
import functools

import jax
import jax.numpy as jnp
from jax import lax
from jax.experimental import pallas as pl
from jax.experimental.pallas import tpu as pltpu
from jax.experimental.pallas import tpu_sc as plsc

H = 128
BE = 2000
NC = 2
NS = 16
NW = NC * NS
SEG = 320
NPAD = NW * SEG
PCH = 8
EBITS = 19
EMASK = (1 << EBITS) - 1


def _vector_mesh():
    return plsc.VectorSubcoreMesh(core_axis_name="c", subcore_axis_name="s")


def _sc_params():
    return pltpu.CompilerParams(needs_layout_passes=False)


def _wid():
    return lax.axis_index("s") * NC + lax.axis_index("c")


def _scalar(v):
    return jnp.max(v)


def _partition(dst):
    E = dst.shape[0]
    CS = E // PCH

    @functools.partial(
        pl.kernel,
        out_type=(jax.ShapeDtypeStruct((NW * PCH * CS,), jnp.int32),
                  jax.ShapeDtypeStruct((NW * 16,), jnp.int32)),
        mesh=_vector_mesh(),
        compiler_params=_sc_params(),
        scratch_types=[pltpu.VMEM((CS,), jnp.int32),
                       pltpu.VMEM((CS + 16,), jnp.int32),
                       pltpu.VMEM((16,), jnp.int32),
                       pltpu.SemaphoreType.DMA],
    )
    def k(dst_hbm, pk_hbm, cnt_hbm, dstbuf, stg, cntv, sem):
        w = _wid()
        lo = w * SEG
        sent = jnp.full((16,), SEG << EBITS, jnp.int32)
        iot = lax.iota(jnp.int32, 16)
        cnts = jnp.zeros((16,), jnp.int32)
        for i in range(PCH):
            pltpu.async_copy(dst_hbm.at[pl.ds(i * CS, CS)], dstbuf, sem).wait()

            def step(kk, off):
                d = dstbuf[pl.ds(kk * 16, 16)]
                m = ((d * 6554) >> 21) == w
                eid = (i * CS + kk * 16) + lax.iota(jnp.int32, 16)
                pk = eid | ((d - lo) << EBITS)
                plsc.store_compressed(stg.at[pl.ds(off, 16)], pk, mask=m)
                return off + plsc.all_reduce_population_count(m)[0]

            off = lax.fori_loop(0, CS // 16, step, 0)
            stg[pl.ds(off, 16)] = sent
            offp = (off + 15) & ~15
            cnts = cnts + jnp.where(iot == i, offp, 0)
            pltpu.async_copy(stg.at[pl.ds(0, CS)],
                             pk_hbm.at[pl.ds((w * PCH + i) * CS, CS)],
                             sem).wait()
        cntv[...] = cnts
        pltpu.async_copy(cntv, cnt_hbm.at[pl.ds(w * 16, 16)], sem).wait()

    return k(dst)


def _sc_scatter_max(vals, pk, cnt, h_prev):
    E = vals.shape[0]
    CS = E // PCH
    GC = 128

    @functools.partial(
        pl.kernel,
        out_type=jax.ShapeDtypeStruct((NPAD, H), jnp.float32),
        mesh=_vector_mesh(),
        compiler_params=_sc_params(),
        scratch_types=[pltpu.VMEM((SEG + 8, H), jnp.float32),
                       pltpu.VMEM((SEG, H), jnp.float32),
                       pltpu.VMEM((GC, H), jnp.float32),
                       pltpu.VMEM((GC, H), jnp.float32),
                       pltpu.VMEM((GC,), jnp.int32),
                       pltpu.VMEM((GC,), jnp.int32),
                       pltpu.VMEM((GC,), jnp.int32),
                       pltpu.VMEM((GC,), jnp.int32),
                       pltpu.VMEM((16,), jnp.int32),
                       pltpu.SemaphoreType.DMA,
                       pltpu.SemaphoreType.DMA,
                       pltpu.SemaphoreType.DMA,
                       pltpu.SemaphoreType.DMA,
                       pltpu.SemaphoreType.DMA],
    )
    def k(vals_hbm, pk_hbm, cnt_hbm, hprev_hbm, hout_hbm,
          acc, hbuf, rows0, rows1, pbuf0, pbuf1, eidx0, eidx1, cvbuf,
          sem, sp0, sp1, sg0, sg1):
        w = _wid()
        neg = jnp.full((16,), -jnp.inf, jnp.float32)

        @pl.loop(0, SEG + 1)
        def _(r):
            for c in range(H // 16):
                acc[r, pl.ds(c * 16, 16)] = neg

        pltpu.async_copy(cnt_hbm.at[pl.ds(w * 16, 16)], cvbuf, sem).wait()
        cv = cvbuf[...]

        def rmw(pbuf, rows, rem):
            def group(g, _):
                dlv = pbuf[pl.ds(g * 16, 16)] >> EBITS
                for e in range(16):
                    dl = dlv[e]
                    for c in range(H // 16):
                        sl = pl.ds(c * 16, 16)
                        er = g * 16 + e
                        acc[dl, sl] = jnp.maximum(acc[dl, sl], rows[er, sl])
                return 0
            lax.fori_loop(0, rem // 16, group, 0)

        for i in range(PCH):
            cnt_i = cv[i]
            base = (w * PCH + i) * CS
            nsub = (cnt_i + GC - 1) // GC

            def fire_pk(j, pbuf, sp):
                pltpu.async_copy(
                    pk_hbm.at[pl.ds(base + j * GC, GC)], pbuf, sp)

            def wait_pk(j, pbuf, sp):
                pltpu.make_async_copy(
                    pk_hbm.at[pl.ds(base + j * GC, GC)], pbuf, sp).wait()

            def mkeidx(pbuf, eidx):
                for t in range(GC // 16):
                    sl = pl.ds(t * 16, 16)
                    eidx[sl] = jnp.minimum(pbuf[sl] & EMASK, E - 1)

            @pl.when(nsub > 0)
            def _():
                fire_pk(0, pbuf0, sp0)

            def pair(t, _):
                j0 = 2 * t
                j1 = 2 * t + 1

                @pl.when(j0 < nsub)
                def _():
                    wait_pk(j0, pbuf0, sp0)
                    mkeidx(pbuf0, eidx0)
                    pltpu.async_copy(vals_hbm.at[eidx0], rows0, sg0)

                @pl.when(j1 < nsub)
                def _():
                    fire_pk(j1, pbuf1, sp1)

                @pl.when(j0 < nsub)
                def _():
                    pltpu.make_async_copy(vals_hbm.at[eidx0], rows0,
                                          sg0).wait()
                    rmw(pbuf0, rows0, jnp.minimum(GC, cnt_i - j0 * GC))

                @pl.when(j1 < nsub)
                def _():
                    wait_pk(j1, pbuf1, sp1)
                    mkeidx(pbuf1, eidx1)
                    pltpu.async_copy(vals_hbm.at[eidx1], rows1, sg1)

                @pl.when(j0 + 2 < nsub)
                def _():
                    fire_pk(j0 + 2, pbuf0, sp0)

                @pl.when(j1 < nsub)
                def _():
                    pltpu.make_async_copy(vals_hbm.at[eidx1], rows1,
                                          sg1).wait()
                    rmw(pbuf1, rows1, jnp.minimum(GC, cnt_i - j1 * GC))

                return 0

            lax.fori_loop(0, (nsub + 1) // 2, pair, 0)

        pltpu.async_copy(hprev_hbm.at[pl.ds(w * SEG, SEG)], hbuf, sem).wait()

        @pl.loop(0, SEG)
        def _(r):
            for c in range(H // 16):
                sl = pl.ds(c * 16, 16)
                a = acc[r, sl]
                hbuf[r, sl] = hbuf[r, sl] + jnp.where(a == -jnp.inf, 0.0, a)

        pltpu.async_copy(hbuf, hout_hbm.at[pl.ds(w * SEG, SEG)], sem).wait()

    return k(vals, pk, cnt, h_prev)


def _sc_gather(h_pad, dst):
    E = dst.shape[0]
    GC = 128
    per_w = E // NW
    nfull = per_w // GC
    tail = per_w - nfull * GC

    npair = nfull // 2

    @functools.partial(
        pl.kernel,
        out_type=jax.ShapeDtypeStruct((E, H), jnp.float32),
        mesh=_vector_mesh(),
        scratch_types=[pltpu.VMEM((GC,), jnp.int32),
                       pltpu.VMEM((GC,), jnp.int32),
                       pltpu.VMEM((tail,), jnp.int32),
                       pltpu.VMEM((GC, H), jnp.float32),
                       pltpu.VMEM((GC, H), jnp.float32),
                       pltpu.VMEM((tail, H), jnp.float32),
                       pltpu.SemaphoreType.DMA,
                       pltpu.SemaphoreType.DMA,
                       pltpu.SemaphoreType.DMA,
                       pltpu.SemaphoreType.DMA,
                       pltpu.SemaphoreType.DMA,
                       pltpu.SemaphoreType.DMA],
    )
    def k(h_hbm, dst_hbm, out_hbm, idx0, idx1, idxt, rows0, rows1, rowst,
          si0, si1, sg0, sg1, so0, so1):
        base = _wid() * per_w

        def fire_idx(j, idx, si):
            pltpu.async_copy(dst_hbm.at[pl.ds(base + j * GC, GC)], idx, si)

        def wait_idx(j, idx, si):
            pltpu.make_async_copy(dst_hbm.at[pl.ds(base + j * GC, GC)],
                                  idx, si).wait()

        fire_idx(0, idx0, si0)
        fire_idx(1, idx1, si1)

        def pair(t, _):
            j0 = 2 * t
            j1 = 2 * t + 1

            @pl.when(t > 0)
            def _():
                pltpu.make_async_copy(
                    rows0, out_hbm.at[pl.ds(base + (j0 - 2) * GC, GC)],
                    so0).wait()
                pltpu.make_async_copy(
                    rows1, out_hbm.at[pl.ds(base + (j1 - 2) * GC, GC)],
                    so1).wait()

            wait_idx(j0, idx0, si0)
            wait_idx(j1, idx1, si1)
            pltpu.async_copy(h_hbm.at[idx0], rows0, sg0)
            pltpu.async_copy(h_hbm.at[idx1], rows1, sg1)

            pltpu.make_async_copy(h_hbm.at[idx0], rows0, sg0).wait()
            pltpu.async_copy(rows0, out_hbm.at[pl.ds(base + j0 * GC, GC)],
                             so0)
            pltpu.make_async_copy(h_hbm.at[idx1], rows1, sg1).wait()
            pltpu.async_copy(rows1, out_hbm.at[pl.ds(base + j1 * GC, GC)],
                             so1)

            @pl.when(t + 1 < npair)
            def _():
                fire_idx(j0 + 2, idx0, si0)
                fire_idx(j1 + 2, idx1, si1)

            return 0

        lax.fori_loop(0, npair, pair, 0)
        pltpu.make_async_copy(
            rows0, out_hbm.at[pl.ds(base + (nfull - 2) * GC, GC)],
            so0).wait()
        pltpu.make_async_copy(
            rows1, out_hbm.at[pl.ds(base + (nfull - 1) * GC, GC)],
            so1).wait()

        if tail:
            o = base + nfull * GC
            pltpu.async_copy(dst_hbm.at[pl.ds(o, tail)], idxt, si0).wait()
            pltpu.async_copy(h_hbm.at[idxt], rowst, sg0).wait()
            pltpu.async_copy(rowst, out_hbm.at[pl.ds(o, tail)], so0).wait()

    return k(h_pad, dst)


def _b2(b):
    return b.reshape(1, -1)


def _embed_fx_body(ea_raw_ref, w1, b1, w2, b2, f1, c1, f2, c2,
                   ea_out, vals_out):
    x = ea_raw_ref[...]
    t = jnp.maximum(jnp.dot(x, w1[...], preferred_element_type=jnp.float32)
                    + b1[...], 0.0)
    ea = jnp.dot(t, w2[...], preferred_element_type=jnp.float32) + b2[...]
    u = jnp.maximum(jnp.dot(ea, f1[...], preferred_element_type=jnp.float32)
                    + c1[...], 0.0)
    vals = jnp.dot(u, f2[...], preferred_element_type=jnp.float32) + c2[...]
    if ea_out is not None:
        ea_out[...] = ea
    vals_out[...] = vals


def _embed_fx(ea_raw, emb, fx, want_ea):
    E = ea_raw.shape[0]
    grid = (E // BE,)
    w_spec = pl.BlockSpec((H, H), lambda i: (0, 0))
    b_spec = pl.BlockSpec((1, H), lambda i: (0, 0))
    in_specs = [
        pl.BlockSpec((BE, 16), lambda i: (i, 0)),
        pl.BlockSpec((16, H), lambda i: (0, 0)), b_spec, w_spec, b_spec,
        w_spec, b_spec, w_spec, b_spec,
    ]
    out_spec = pl.BlockSpec((BE, H), lambda i: (i, 0))
    if want_ea:
        out_shape = (jax.ShapeDtypeStruct((E, H), jnp.float32),
                     jax.ShapeDtypeStruct((E, H), jnp.float32))
        fn = pl.pallas_call(
            lambda *refs: _embed_fx_body(*refs[:9], refs[9], refs[10]),
            grid=grid, in_specs=in_specs,
            out_specs=(out_spec, out_spec), out_shape=out_shape)
    else:
        out_shape = jax.ShapeDtypeStruct((E, H), jnp.float32)
        fn = pl.pallas_call(
            lambda *refs: _embed_fx_body(*refs[:9], None, refs[9]),
            grid=grid, in_specs=in_specs,
            out_specs=out_spec, out_shape=out_shape)
    return fn(ea_raw, emb["W1"], _b2(emb["b1"]), emb["W2"], _b2(emb["b2"]),
              fx["W1"], _b2(fx["b1"]), fx["W2"], _b2(fx["b2"]))


def _update_fx_body(ea_ref, hg_ref, m1a, m1b, bm1, m2, bm2, f1, c1, f2, c2,
                    ea_out, vals_out):
    ea = ea_ref[...]
    hg = hg_ref[...]
    t = jnp.maximum(
        jnp.dot(ea, m1a[...], preferred_element_type=jnp.float32)
        + jnp.dot(hg, m1b[...], preferred_element_type=jnp.float32)
        + bm1[...], 0.0)
    ea2 = ea + jnp.dot(t, m2[...], preferred_element_type=jnp.float32) + bm2[...]
    u = jnp.maximum(jnp.dot(ea2, f1[...], preferred_element_type=jnp.float32)
                    + c1[...], 0.0)
    vals = jnp.dot(u, f2[...], preferred_element_type=jnp.float32) + c2[...]
    if ea_out is not None:
        ea_out[...] = ea2
    vals_out[...] = vals


def _update_fx(ea, hg, mlp, fx, want_ea):
    E = ea.shape[0]
    grid = (E // BE,)
    w_spec = pl.BlockSpec((H, H), lambda i: (0, 0))
    b_spec = pl.BlockSpec((1, H), lambda i: (0, 0))
    e_spec = pl.BlockSpec((BE, H), lambda i: (i, 0))
    in_specs = [e_spec, e_spec,
                w_spec, w_spec, b_spec, w_spec, b_spec,
                w_spec, b_spec, w_spec, b_spec]
    m1a = mlp["W1"][:H]
    m1b = mlp["W1"][H:]
    if want_ea:
        out_shape = (jax.ShapeDtypeStruct((E, H), jnp.float32),
                     jax.ShapeDtypeStruct((E, H), jnp.float32))
        fn = pl.pallas_call(
            lambda *refs: _update_fx_body(*refs[:11], refs[11], refs[12]),
            grid=grid, in_specs=in_specs,
            out_specs=(e_spec, e_spec), out_shape=out_shape)
    else:
        out_shape = jax.ShapeDtypeStruct((E, H), jnp.float32)
        fn = pl.pallas_call(
            lambda *refs: _update_fx_body(*refs[:11], None, refs[11]),
            grid=grid, in_specs=in_specs,
            out_specs=e_spec, out_shape=out_shape)
    return fn(ea, hg, m1a, m1b, _b2(mlp["b1"]), mlp["W2"], _b2(mlp["b2"]),
              fx["W1"], _b2(fx["b1"]), fx["W2"], _b2(fx["b2"]))


def _field_body(vec_ref, act_ref, w1v, w1a, b1, w2, b2, out_ref):
    t = jnp.maximum(
        jnp.dot(vec_ref[...], w1v[...], preferred_element_type=jnp.float32)
        + jnp.dot(act_ref[...], w1a[...], preferred_element_type=jnp.float32)
        + b1[...], 0.0)
    out_ref[...] = jnp.dot(t, w2[...], preferred_element_type=jnp.float32) + b2[...]


def _field(vec, action, p):
    n = vec.shape[0]
    w1v = p["W1"][:H]
    w1a = p["W1"][H:]
    fn = pl.pallas_call(
        _field_body,
        out_shape=jax.ShapeDtypeStruct((n, 1), jnp.float32))
    return fn(vec, action, w1v, w1a, _b2(p["b1"]), p["W2"],
              _b2(p["b2"])).squeeze(-1)


def _process_type(ea_raw, dst, p):
    n_layers = len(p["layers"])
    pk, cnt = _partition(dst)
    want_ea = n_layers > 1
    res = _embed_fx(ea_raw, p["embed"], p["layers"][0]["fx"], want_ea)
    if want_ea:
        ea, vals = res
    else:
        vals = res
    h = _sc_scatter_max(vals, pk, cnt,
                        jnp.zeros((NPAD, H), jnp.float32))
    for li in range(1, n_layers):
        hg = _sc_gather(h, dst)
        last = li == n_layers - 1
        res = _update_fx(ea, hg, p["layers"][li - 1]["mlp"],
                         p["layers"][li]["fx"], not last)
        if not last:
            ea, vals = res
        else:
            vals = res
        h = _sc_scatter_max(vals, pk, cnt, h)
    return h


def kernel(x_obstacle, x_agent, x_goal, ei_ona, ei_ana, ei_tow,
           ea_ona, ea_ana, ea_tow, action, params):
    n = x_agent.shape[0]
    h_ona = _process_type(ea_ona, ei_ona[1].astype(jnp.int32), params["ona"])
    h_ana = _process_type(ea_ana, ei_ana[1].astype(jnp.int32), params["ana"])
    h_tow = _process_type(ea_tow, ei_tow[1].astype(jnp.int32), params["tow"])
    vec = jnp.maximum(jnp.maximum(jnp.maximum(h_ona, h_ana), h_tow),
                      0.0)[:n]
    return _field(vec, action, params["field"])
